# baseline (speedup 1.0000x reference)
import jax
import jax.numpy as jnp
from jax import lax
from jax.experimental import pallas as pl
from jax.experimental.pallas import tpu as pltpu
from jax.experimental.pallas import tpu_sc as plsc

NT = 100000
NC = 2
NS = 16
L = 16
NW = NC * NS
B = 16384
D = 64
W = 2 * D
BPW = B // NW
CHUNK = 128
NCH = BPW // CHUNK
GPC = CHUNK // L

RB = 4000
PACK_GRID = NT // RB


def _pack_body(up_ref, iq_ref, out_ref):
    out_ref[...] = jnp.concatenate([up_ref[...], iq_ref[...]], axis=1)


def _tc_pack(user_p, item_q):
    return pl.pallas_call(
        _pack_body,
        grid=(PACK_GRID,),
        in_specs=[
            pl.BlockSpec((RB, D), lambda g: (g, 0)),
            pl.BlockSpec((RB, D), lambda g: (g, 0)),
        ],
        out_specs=pl.BlockSpec((RB, W), lambda g: (g, 0)),
        out_shape=jax.ShapeDtypeStruct((NT, W), jnp.float32),
    )(user_p, item_q)


def _mf_body(u_hbm, i_hbm, cat_hbm, ub_hbm, ib_hbm, mu_hbm, out_hbm,
             uidx, iidx, up_rows, iq_rows, ubv, ibv, outv, muv,
             bsem, gsem0, gsem1):
    c = lax.axis_index("c")
    s = lax.axis_index("s")
    wid = s * NC + c
    base = wid * BPW

    pltpu.sync_copy(u_hbm.at[pl.ds(base, BPW)], uidx)
    pltpu.sync_copy(i_hbm.at[pl.ds(base, BPW)], iidx)
    pltpu.sync_copy(mu_hbm, muv)

    bias_copies = [
        pltpu.async_copy(ub_hbm.at[uidx], ubv, bsem),
        pltpu.async_copy(ib_hbm.at[iidx], ibv, bsem),
    ]

    mu_s = muv[...]
    lane = lax.broadcasted_iota(jnp.int32, (L,), 0)
    gsems = (gsem0, gsem1)

    def fire(ci):
        p = ci % 2
        sl = pl.ds(ci * CHUNK, CHUNK)
        return (
            pltpu.async_copy(cat_hbm.at[uidx.at[sl]], up_rows.at[p], gsems[p]),
            pltpu.async_copy(cat_hbm.at[iidx.at[sl]], iq_rows.at[p], gsems[p]),
        )

    def compute(ci):
        p = ci % 2
        upb = up_rows.at[p]
        iqb = iq_rows.at[p]

        def group_body(g, carry):
            rows = lane + g * L

            def d_body(d, acc):
                dv = jnp.full((L,), d, jnp.int32)
                upv = plsc.load_gather(upb, [rows, dv])
                iqv = plsc.load_gather(iqb, [rows, dv + D])
                return acc + upv * iqv

            acc = lax.fori_loop(0, D, d_body, jnp.zeros((L,), jnp.float32),
                                unroll=8)
            osl = pl.ds(ci * CHUNK + g * L, L)
            outv[osl] = acc + ubv[osl] + ibv[osl] + mu_s
            return carry

        lax.fori_loop(0, GPC, group_body, 0)

    inflight = fire(0)
    for ci in range(NCH):
        nxt = fire(ci + 1) if ci + 1 < NCH else None
        for cp in inflight:
            cp.wait()
        compute(ci)
        inflight = nxt

    for cp in bias_copies:
        cp.wait()
    pltpu.sync_copy(outv, out_hbm.at[pl.ds(base, BPW)])


@jax.jit
def kernel(u, i, user_p, item_q, user_b, item_b, mu):
    cat = _tc_pack(user_p, item_q)
    ub1 = user_b.reshape(-1)
    ib1 = item_b.reshape(-1)
    mu16 = jnp.broadcast_to(mu, (L,))
    mesh = plsc.VectorSubcoreMesh(core_axis_name="c", subcore_axis_name="s",
                                  num_cores=NC, num_subcores=NS)
    fn = pl.kernel(
        _mf_body,
        out_type=jax.ShapeDtypeStruct((B,), jnp.float32),
        mesh=mesh,
        compiler_params=pltpu.CompilerParams(needs_layout_passes=False),
        scratch_types=[
            pltpu.VMEM((BPW,), jnp.int32),
            pltpu.VMEM((BPW,), jnp.int32),
            pltpu.VMEM((2, CHUNK, W), jnp.float32),
            pltpu.VMEM((2, CHUNK, W), jnp.float32),
            pltpu.VMEM((BPW,), jnp.float32),
            pltpu.VMEM((BPW,), jnp.float32),
            pltpu.VMEM((BPW,), jnp.float32),
            pltpu.VMEM((L,), jnp.float32),
            pltpu.SemaphoreType.DMA,
            pltpu.SemaphoreType.DMA,
            pltpu.SemaphoreType.DMA,
        ],
    )
    return fn(u, i, cat, ub1, ib1, mu16)


# speedup vs baseline: 1.0121x; 1.0121x over previous
import jax
import jax.numpy as jnp
from jax import lax
from jax.experimental import pallas as pl
from jax.experimental.pallas import tpu as pltpu
from jax.experimental.pallas import tpu_sc as plsc

NT = 100000
NC = 2
NS = 16
L = 16
NW = NC * NS
B = 16384
D = 64
W = 2 * D
BPW = B // NW
CHUNK = 128
NCH = BPW // CHUNK
GPC = CHUNK // L

RB = 10000
PACK_GRID = NT // RB


def _pack_body(up_ref, iq_ref, out_ref):
    out_ref[:, 0:D] = up_ref[...]
    out_ref[:, D:W] = iq_ref[...]


def _tc_pack(user_p, item_q):
    return pl.pallas_call(
        _pack_body,
        grid=(PACK_GRID,),
        in_specs=[
            pl.BlockSpec((RB, D), lambda g: (g, 0)),
            pl.BlockSpec((RB, D), lambda g: (g, 0)),
        ],
        out_specs=pl.BlockSpec((RB, W), lambda g: (g, 0)),
        out_shape=jax.ShapeDtypeStruct((NT, W), jnp.float32),
    )(user_p, item_q)


def _mf_body(u_hbm, i_hbm, cat_hbm, ub_hbm, ib_hbm, mu_hbm, out_hbm,
             uidx, iidx, up_rows, iq_rows, ubv, ibv, outv, muv,
             bsem, gsem0, gsem1):
    c = lax.axis_index("c")
    s = lax.axis_index("s")
    wid = s * NC + c
    base = wid * BPW

    pltpu.sync_copy(u_hbm.at[pl.ds(base, BPW)], uidx)
    pltpu.sync_copy(i_hbm.at[pl.ds(base, BPW)], iidx)
    pltpu.sync_copy(mu_hbm, muv)

    bias_copies = [
        pltpu.async_copy(ub_hbm.at[uidx], ubv, bsem),
        pltpu.async_copy(ib_hbm.at[iidx], ibv, bsem),
    ]

    mu_s = muv[...]
    lane = lax.broadcasted_iota(jnp.int32, (L,), 0)
    gsems = (gsem0, gsem1)

    def fire(ci):
        p = ci % 2
        sl = pl.ds(ci * CHUNK, CHUNK)
        return (
            pltpu.async_copy(cat_hbm.at[uidx.at[sl]], up_rows.at[p], gsems[p]),
            pltpu.async_copy(cat_hbm.at[iidx.at[sl]], iq_rows.at[p], gsems[p]),
        )

    def compute(ci):
        p = ci % 2
        upb = up_rows.at[p]
        iqb = iq_rows.at[p]

        def group_body(g, carry):
            rows = lane + g * L

            def d_body(d, acc):
                dv = jnp.full((L,), d, jnp.int32)
                upv = plsc.load_gather(upb, [rows, dv])
                iqv = plsc.load_gather(iqb, [rows, dv + D])
                return acc + upv * iqv

            acc = lax.fori_loop(0, D, d_body, jnp.zeros((L,), jnp.float32),
                                unroll=8)
            osl = pl.ds(ci * CHUNK + g * L, L)
            outv[osl] = acc + ubv[osl] + ibv[osl] + mu_s
            return carry

        lax.fori_loop(0, GPC, group_body, 0)

    inflight = fire(0)
    for ci in range(NCH):
        nxt = fire(ci + 1) if ci + 1 < NCH else None
        for cp in inflight:
            cp.wait()
        compute(ci)
        inflight = nxt

    for cp in bias_copies:
        cp.wait()
    pltpu.sync_copy(outv, out_hbm.at[pl.ds(base, BPW)])


@jax.jit
def kernel(u, i, user_p, item_q, user_b, item_b, mu):
    cat = _tc_pack(user_p, item_q)
    ub1 = user_b.reshape(-1)
    ib1 = item_b.reshape(-1)
    mu16 = jnp.broadcast_to(mu, (L,))
    mesh = plsc.VectorSubcoreMesh(core_axis_name="c", subcore_axis_name="s",
                                  num_cores=NC, num_subcores=NS)
    fn = pl.kernel(
        _mf_body,
        out_type=jax.ShapeDtypeStruct((B,), jnp.float32),
        mesh=mesh,
        compiler_params=pltpu.CompilerParams(needs_layout_passes=False),
        scratch_types=[
            pltpu.VMEM((BPW,), jnp.int32),
            pltpu.VMEM((BPW,), jnp.int32),
            pltpu.VMEM((2, CHUNK, W), jnp.float32),
            pltpu.VMEM((2, CHUNK, W), jnp.float32),
            pltpu.VMEM((BPW,), jnp.float32),
            pltpu.VMEM((BPW,), jnp.float32),
            pltpu.VMEM((BPW,), jnp.float32),
            pltpu.VMEM((L,), jnp.float32),
            pltpu.SemaphoreType.DMA,
            pltpu.SemaphoreType.DMA,
            pltpu.SemaphoreType.DMA,
        ],
    )
    return fn(u, i, cat, ub1, ib1, mu16)


# speedup vs baseline: 1.2024x; 1.1881x over previous
import jax
import jax.numpy as jnp
from jax import lax
from jax.experimental import pallas as pl
from jax.experimental.pallas import tpu as pltpu
from jax.experimental.pallas import tpu_sc as plsc

NT = 100000
NC = 2
NS = 16
L = 16
NW = NC * NS
B = 16384
D = 64
W = 2 * D
BPW = B // NW
CHUNK = 128
NCH = BPW // CHUNK
GPC = CHUNK // L


def _mf_body(u_hbm, i_hbm, cat_hbm, ub_hbm, ib_hbm, mu_hbm, out_hbm,
             uidx, iidx, up_rows, iq_rows, ubv, ibv, outv, muv,
             bsem, gsem0, gsem1):
    c = lax.axis_index("c")
    s = lax.axis_index("s")
    wid = s * NC + c
    base = wid * BPW

    pltpu.sync_copy(u_hbm.at[pl.ds(base, BPW)], uidx)
    pltpu.sync_copy(i_hbm.at[pl.ds(base, BPW)], iidx)
    pltpu.sync_copy(mu_hbm, muv)

    bias_copies = []
    for ci in range(NCH):
        sl = pl.ds(ci * CHUNK, CHUNK)
        bias_copies.append(
            pltpu.async_copy(ub_hbm.at[uidx.at[sl]], ubv.at[sl], bsem))
        bias_copies.append(
            pltpu.async_copy(ib_hbm.at[iidx.at[sl]], ibv.at[sl], bsem))

    mu_s = muv[...]
    lane = lax.broadcasted_iota(jnp.int32, (L,), 0)
    gsems = (gsem0, gsem1)

    def fire(ci):
        p = ci % 2
        sl = pl.ds(ci * CHUNK, CHUNK)
        return (
            pltpu.async_copy(cat_hbm.at[uidx.at[sl]], up_rows.at[p], gsems[p]),
            pltpu.async_copy(cat_hbm.at[iidx.at[sl]], iq_rows.at[p], gsems[p]),
        )

    def compute(ci):
        p = ci % 2
        upb = up_rows.at[p]
        iqb = iq_rows.at[p]

        def group_body(g, carry):
            rows = lane + g * L

            def d_body(d, acc):
                dv = jnp.full((L,), d, jnp.int32)
                upv = plsc.load_gather(upb, [rows, dv])
                iqv = plsc.load_gather(iqb, [rows, dv + D])
                return acc + upv * iqv

            acc = lax.fori_loop(0, D, d_body, jnp.zeros((L,), jnp.float32),
                                unroll=8)
            osl = pl.ds(ci * CHUNK + g * L, L)
            outv[osl] = acc + ubv[osl] + ibv[osl] + mu_s
            return carry

        lax.fori_loop(0, GPC, group_body, 0)

    inflight = fire(0)
    for ci in range(NCH):
        nxt = fire(ci + 1) if ci + 1 < NCH else None
        for cp in inflight:
            cp.wait()
        compute(ci)
        inflight = nxt

    for cp in bias_copies:
        cp.wait()
    pltpu.sync_copy(outv, out_hbm.at[pl.ds(base, BPW)])


@jax.jit
def kernel(u, i, user_p, item_q, user_b, item_b, mu):
    cat = jnp.concatenate([user_p, item_q], axis=1)
    ub1 = user_b.reshape(-1)
    ib1 = item_b.reshape(-1)
    mu16 = jnp.broadcast_to(mu, (L,))
    mesh = plsc.VectorSubcoreMesh(core_axis_name="c", subcore_axis_name="s",
                                  num_cores=NC, num_subcores=NS)
    fn = pl.kernel(
        _mf_body,
        out_type=jax.ShapeDtypeStruct((B,), jnp.float32),
        mesh=mesh,
        compiler_params=pltpu.CompilerParams(needs_layout_passes=False),
        scratch_types=[
            pltpu.VMEM((BPW,), jnp.int32),
            pltpu.VMEM((BPW,), jnp.int32),
            pltpu.VMEM((2, CHUNK, W), jnp.float32),
            pltpu.VMEM((2, CHUNK, W), jnp.float32),
            pltpu.VMEM((BPW,), jnp.float32),
            pltpu.VMEM((BPW,), jnp.float32),
            pltpu.VMEM((BPW,), jnp.float32),
            pltpu.VMEM((L,), jnp.float32),
            pltpu.SemaphoreType.DMA,
            pltpu.SemaphoreType.DMA,
            pltpu.SemaphoreType.DMA,
        ],
    )
    return fn(u, i, cat, ub1, ib1, mu16)


# speedup vs baseline: 1.2035x; 1.0009x over previous
import jax
import jax.numpy as jnp
from jax import lax
from jax.experimental import pallas as pl
from jax.experimental.pallas import tpu as pltpu
from jax.experimental.pallas import tpu_sc as plsc

NT = 100000
NC = 2
NS = 16
L = 16
NW = NC * NS
B = 16384
D = 64
W = 2 * D
BPW = B // NW
CHUNK = 128
NCH = BPW // CHUNK
GPC = CHUNK // L


def _mf_body(u_hbm, i_hbm, cat_hbm, ub_hbm, ib_hbm, mu_hbm, out_hbm,
             uidx, iidx, up_rows, iq_rows, ubv, ibv, outv, muv,
             bsem, gsem0, gsem1):
    c = lax.axis_index("c")
    s = lax.axis_index("s")
    wid = s * NC + c
    base = wid * BPW

    pltpu.sync_copy(u_hbm.at[pl.ds(base, BPW)], uidx)
    pltpu.sync_copy(i_hbm.at[pl.ds(base, BPW)], iidx)
    pltpu.sync_copy(mu_hbm, muv)

    bias_copies = []
    for ci in range(NCH):
        sl = pl.ds(ci * CHUNK, CHUNK)
        bias_copies.append(
            pltpu.async_copy(ub_hbm.at[uidx.at[sl]], ubv.at[sl], bsem))
        bias_copies.append(
            pltpu.async_copy(ib_hbm.at[iidx.at[sl]], ibv.at[sl], bsem))

    mu_s = muv[...]
    lane = lax.broadcasted_iota(jnp.int32, (L,), 0)
    gsems = (gsem0, gsem1)

    def fire(ci):
        p = ci % 2
        sl = pl.ds(ci * CHUNK, CHUNK)
        return (
            pltpu.async_copy(cat_hbm.at[uidx.at[sl]], up_rows.at[p], gsems[p]),
            pltpu.async_copy(cat_hbm.at[iidx.at[sl]], iq_rows.at[p], gsems[p]),
        )

    def compute(ci):
        p = ci % 2
        upb = up_rows.at[p]
        iqb = iq_rows.at[p]

        def group_body(g, carry):
            rows = lane + g * L

            def d_body(d, acc):
                dv = jnp.full((L,), d, jnp.int32)
                upv = plsc.load_gather(upb, [rows, dv])
                iqv = plsc.load_gather(iqb, [rows, dv + D])
                return acc + upv * iqv

            acc = lax.fori_loop(0, D, d_body, jnp.zeros((L,), jnp.float32),
                                unroll=16)
            osl = pl.ds(ci * CHUNK + g * L, L)
            outv[osl] = acc + ubv[osl] + ibv[osl] + mu_s
            return carry

        lax.fori_loop(0, GPC, group_body, 0)

    inflight = fire(0)
    for ci in range(NCH):
        nxt = fire(ci + 1) if ci + 1 < NCH else None
        for cp in inflight:
            cp.wait()
        compute(ci)
        inflight = nxt

    for cp in bias_copies:
        cp.wait()
    pltpu.sync_copy(outv, out_hbm.at[pl.ds(base, BPW)])


@jax.jit
def kernel(u, i, user_p, item_q, user_b, item_b, mu):
    cat = jnp.concatenate([user_p, item_q], axis=1)
    ub1 = user_b.reshape(-1)
    ib1 = item_b.reshape(-1)
    mu16 = jnp.broadcast_to(mu, (L,))
    mesh = plsc.VectorSubcoreMesh(core_axis_name="c", subcore_axis_name="s",
                                  num_cores=NC, num_subcores=NS)
    fn = pl.kernel(
        _mf_body,
        out_type=jax.ShapeDtypeStruct((B,), jnp.float32),
        mesh=mesh,
        compiler_params=pltpu.CompilerParams(needs_layout_passes=False),
        scratch_types=[
            pltpu.VMEM((BPW,), jnp.int32),
            pltpu.VMEM((BPW,), jnp.int32),
            pltpu.VMEM((2, CHUNK, W), jnp.float32),
            pltpu.VMEM((2, CHUNK, W), jnp.float32),
            pltpu.VMEM((BPW,), jnp.float32),
            pltpu.VMEM((BPW,), jnp.float32),
            pltpu.VMEM((BPW,), jnp.float32),
            pltpu.VMEM((L,), jnp.float32),
            pltpu.SemaphoreType.DMA,
            pltpu.SemaphoreType.DMA,
            pltpu.SemaphoreType.DMA,
        ],
    )
    return fn(u, i, cat, ub1, ib1, mu16)
